# baseline (speedup 1.0000x reference)
import functools

import jax
import jax.numpy as jnp
from jax.experimental import pallas as pl
from jax.experimental.pallas import tpu as pltpu

H = 4096
H4 = H // 4
BT = 1024
NCHUNK = 8
NCORE = 2
WCHUNK = 256


def _body(layer_idx_ref, x_ref, w_hbm, b1_ref, w2_ref, b2_ref, lb_ref,
          o_ref, w8_ref, wtmp_ref, dma_sem):
    @pl.when(pl.program_id(1) == 0)
    def _():
        for i in range(H4 // WCHUNK):
            copy = pltpu.make_async_copy(
                w_hbm.at[pl.ds(i * WCHUNK, WCHUNK), :], wtmp_ref, dma_sem)
            copy.start()
            copy.wait()
            w8_ref[:, pl.ds(i * WCHUNK, WCHUNK)] = (
                wtmp_ref[...].T.astype(jnp.bfloat16))

    w8 = w8_ref[...]
    wd = (w2_ref[1, :] - w2_ref[0, :]).reshape(1, H4)
    c = b2_ref[1] - b2_ref[0] + lb_ref[layer_idx_ref[0]]
    b1 = b1_ref[...]
    mc = BT // NCHUNK
    for j in range(NCHUNK):
        xb = x_ref[pl.ds(j * mc, mc), :].astype(jnp.bfloat16)
        h = jax.lax.dot_general(
            xb, w8, (((1,), (0,)), ((), ())),
            preferred_element_type=jnp.float32,
        )
        h = h + b1
        h = h * jax.nn.sigmoid(h)
        t = jnp.sum(h * wd, axis=1) + c
        o_ref[0, 0, pl.ds(j * mc, mc)] = jax.nn.sigmoid(t)


@functools.partial(jax.jit, static_argnames=())
def kernel(hidden_states, layer_idx, W1, b1, W2, b2, layer_bias):
    orig_shape = hidden_states.shape[:-1]
    x = hidden_states.reshape(-1, H)
    n = x.shape[0]
    nb = n // BT
    npc = nb // NCORE

    out = pl.pallas_call(
        _body,
        grid=(NCORE, npc),
        in_specs=[
            pl.BlockSpec(memory_space=pltpu.SMEM),
            pl.BlockSpec((BT, H), lambda i, k: (i * npc + k, 0)),
            pl.BlockSpec(memory_space=pl.ANY),
            pl.BlockSpec((1, H4), lambda i, k: (0, 0)),
            pl.BlockSpec((2, H4), lambda i, k: (0, 0)),
            pl.BlockSpec(memory_space=pltpu.SMEM),
            pl.BlockSpec(memory_space=pltpu.SMEM),
        ],
        out_specs=pl.BlockSpec((1, 1, BT), lambda i, k: (i * npc + k, 0, 0)),
        out_shape=jax.ShapeDtypeStruct((nb, 1, BT), jnp.float32),
        scratch_shapes=[
            pltpu.VMEM((H, H4), jnp.bfloat16),
            pltpu.VMEM((WCHUNK, H), jnp.float32),
            pltpu.SemaphoreType.DMA,
        ],
        compiler_params=pltpu.CompilerParams(
            dimension_semantics=("parallel", "arbitrary"),
        ),
    )(jnp.reshape(layer_idx, (1,)).astype(jnp.int32), x, W1,
      b1.reshape(1, H4), W2, b2, layer_bias)
    return out.reshape(orig_shape)


# speedup vs baseline: 1.0928x; 1.0928x over previous
import functools

import jax
import jax.numpy as jnp
from jax.experimental import pallas as pl
from jax.experimental.pallas import tpu as pltpu

H = 4096
H4 = H // 4
BT = 1024
NCHUNK = 8


def _body(layer_idx_ref, x_ref, w_ref, b1_ref, w2_ref, b2_ref, lb_ref, o_ref):
    w = w_ref[...]
    b1 = b1_ref[...]
    wd = (w2_ref[1:2, :] - w2_ref[0:1, :])
    c = b2_ref[1] - b2_ref[0] + lb_ref[layer_idx_ref[0]]
    mc = BT // NCHUNK
    for j in range(NCHUNK):
        xb = x_ref[pl.ds(j * mc, mc), :].astype(jnp.bfloat16)
        h = jax.lax.dot_general(
            xb, w, (((1,), (0,)), ((), ())),
            preferred_element_type=jnp.float32,
        )
        h = h + b1
        h = h * jax.nn.sigmoid(h)
        t = jnp.sum(h * wd, axis=1) + c
        o_ref[0, 0, pl.ds(j * mc, mc)] = jax.nn.sigmoid(t)


@functools.partial(jax.jit, static_argnames=())
def kernel(hidden_states, layer_idx, W1, b1, W2, b2, layer_bias):
    orig_shape = hidden_states.shape[:-1]
    x = hidden_states.reshape(-1, H)
    n = x.shape[0]
    nb = n // BT

    w1t = W1.T.astype(jnp.bfloat16)

    out = pl.pallas_call(
        _body,
        grid=(nb,),
        in_specs=[
            pl.BlockSpec(memory_space=pltpu.SMEM),
            pl.BlockSpec((BT, H), lambda i: (i, 0)),
            pl.BlockSpec((H, H4), lambda i: (0, 0)),
            pl.BlockSpec((1, H4), lambda i: (0, 0)),
            pl.BlockSpec((2, H4), lambda i: (0, 0)),
            pl.BlockSpec(memory_space=pltpu.SMEM),
            pl.BlockSpec(memory_space=pltpu.SMEM),
        ],
        out_specs=pl.BlockSpec((1, 1, BT), lambda i: (i, 0, 0)),
        out_shape=jax.ShapeDtypeStruct((nb, 1, BT), jnp.float32),
        compiler_params=pltpu.CompilerParams(
            dimension_semantics=("parallel",),
        ),
    )(jnp.reshape(layer_idx, (1,)).astype(jnp.int32), x, w1t,
      b1.reshape(1, H4), W2, b2, layer_bias)
    return out.reshape(orig_shape)
